# baseline (speedup 1.0000x reference)
import functools

import jax
import jax.numpy as jnp
from jax import lax
from jax.experimental import pallas as pl
from jax.experimental.pallas import tpu as pltpu
from jax.experimental.pallas import tpu_sc as plsc

_TOP_N = 5000
_BLK = 1024
_NBLK = 5
_PAD_N = _BLK * _NBLK
_THR = 0.7

_SORT_N = 32768
_SORT_R = _SORT_N // 128
_TOPR = _PAD_N // 128


def _sort_body(key_ref, ko_ref, io_ref):
    row = lax.broadcasted_iota(jnp.int32, (_SORT_R, 128), 0)
    lane = lax.broadcasted_iota(jnp.int32, (_SORT_R, 128), 1)
    k = key_ref[...]
    v = row * 128 + lane

    def xor_rows(a, dr):
        g = _SORT_R // (2 * dr)
        a4 = a.reshape(g, 2, dr, 128)
        return jnp.concatenate([a4[:, 1:2], a4[:, 0:1]], axis=1).reshape(_SORT_R, 128)

    def xor_lanes(a, d, sel):
        return jnp.where(sel, pltpu.roll(a, 128 - d, 1), pltpu.roll(a, d, 1))

    for s in range(1, 16):
        m = 1 << s
        dirm = ((lane & m) == 0) if m <= 64 else ((row & (m >> 7)) == 0)
        d = m >> 1
        while d >= 1:
            if d >= 128:
                dr = d >> 7
                pk = xor_rows(k, dr)
                pv = xor_rows(v, dr)
                low = (row & dr) == 0
            else:
                sel = (lane & d) == 0
                pk = xor_lanes(k, d, sel)
                pv = xor_lanes(v, d, sel)
                low = sel
            self_first = (k > pk) | ((k == pk) & (v < pv))
            take_self = self_first == (low == dirm)
            k = jnp.where(take_self, k, pk)
            v = jnp.where(take_self, v, pv)
            d >>= 1
    ko_ref[...] = k[:_TOPR, :]
    io_ref[...] = v[:_TOPR, :]


_sort_call = pl.pallas_call(
    _sort_body,
    out_shape=(
        jax.ShapeDtypeStruct((_TOPR, 128), jnp.int32),
        jax.ShapeDtypeStruct((_TOPR, 128), jnp.int32),
    ),
)


def _col(v, eye):
    return lax.dot_general(eye, v, (((1,), (1,)), ((), ())),
                           preferred_element_type=jnp.float32,
                           precision=lax.Precision.HIGHEST)


def _iou(x1a, y1a, x2a, y2a, aa, x1b, y1b, x2b, y2b, ab):
    ix1 = jnp.maximum(x1a, x1b)
    iy1 = jnp.maximum(y1a, y1b)
    ix2 = jnp.minimum(x2a, x2b)
    iy2 = jnp.minimum(y2a, y2b)
    iw = jnp.maximum(ix2 - ix1, 0.0)
    ih = jnp.maximum(iy2 - iy1, 0.0)
    inter = iw * ih
    union = aa + ab - inter
    return inter / (union + 1e-8)


def _nms_body(rawt_ref, sc_ref, out_ref, c_ref, keep_ref):
    r128 = lax.broadcasted_iota(jnp.int32, (_BLK, _BLK), 0)
    c128 = lax.broadcasted_iota(jnp.int32, (_BLK, _BLK), 1)
    eye = jnp.where(r128 == c128, 1.0, 0.0).astype(jnp.float32)
    upper = r128 < c128

    rx = rawt_ref[0:1, :]
    ry = rawt_ref[1:2, :]
    rw = rawt_ref[2:3, :]
    rh = rawt_ref[3:4, :]
    cx = rx * 1000.0
    cy = ry * 1000.0
    w = rw * 200.0 + 1.0
    h = rh * 200.0 + 1.0
    c_ref[0:1, :] = cx - 0.5 * w
    c_ref[1:2, :] = cy - 0.5 * h
    c_ref[2:3, :] = cx + 0.5 * w
    c_ref[3:4, :] = cy + 0.5 * h
    c_ref[4:5, :] = (c_ref[2:3, :] - c_ref[0:1, :]) * (c_ref[3:4, :] - c_ref[1:2, :])
    keep_ref[...] = jnp.ones((1, _PAD_N), jnp.float32)

    for i in range(_NBLK):
        base = i * _BLK
        x1r = c_ref[0:1, base:base + _BLK]
        y1r = c_ref[1:2, base:base + _BLK]
        x2r = c_ref[2:3, base:base + _BLK]
        y2r = c_ref[3:4, base:base + _BLK]
        arr = c_ref[4:5, base:base + _BLK]
        x1c = _col(x1r, eye)
        y1c = _col(y1r, eye)
        x2c = _col(x2r, eye)
        y2c = _col(y2r, eye)
        arc = _col(arr, eye)

        iou_ii = _iou(x1c, y1c, x2c, y2c, arc, x1r, y1r, x2r, y2r, arr)
        E = jnp.where((iou_ii > _THR) & upper, 1.0, 0.0).astype(jnp.float32)
        inc = keep_ref[0:1, base:base + _BLK]

        def _cond(carry):
            return carry[1]

        def _body(carry):
            k = carry[0]
            cnt = lax.dot_general(k, E, (((1,), (0,)), ((), ())),
                                  preferred_element_type=jnp.float32)
            knew = jnp.where(cnt > 0.5, 0.0, inc)
            changed = jnp.sum(jnp.abs(knew - k)) > 0.0
            return (knew, changed)

        ki, _ = lax.while_loop(_cond, _body, (inc, jnp.asarray(True)))
        keep_ref[0:1, base:base + _BLK] = ki

        if i + 1 < _NBLK:
            tail = base + _BLK
            iou_cross = _iou(x1c, y1c, x2c, y2c, arc,
                             c_ref[0:1, tail:], c_ref[1:2, tail:],
                             c_ref[2:3, tail:], c_ref[3:4, tail:],
                             c_ref[4:5, tail:])
            M = jnp.where(iou_cross > _THR, 1.0, 0.0).astype(jnp.float32)
            cnt = lax.dot_general(ki, M, (((1,), (0,)), ((), ())),
                                  preferred_element_type=jnp.float32)
            keep_ref[0:1, tail:] = jnp.where(cnt > 0.5, 0.0,
                                             keep_ref[0:1, tail:])

    k = keep_ref[...]
    o8 = jnp.concatenate([
        c_ref[0:4, :] * k,
        sc_ref[...] * k,
        jnp.zeros((3, _PAD_N), jnp.float32),
    ], axis=0)
    for i in range(_NBLK):
        base = i * _BLK
        out_ref[base:base + _BLK, :] = lax.dot_general(
            eye, o8[:, base:base + _BLK], (((1,), (1,)), ((), ())),
            preferred_element_type=jnp.float32,
            precision=lax.Precision.HIGHEST)


_nms_call = pl.pallas_call(
    _nms_body,
    out_shape=jax.ShapeDtypeStruct((_PAD_N, 8), jnp.float32),
    scratch_shapes=[
        pltpu.VMEM((8, _PAD_N), jnp.float32),
        pltpu.VMEM((1, _PAD_N), jnp.float32),
    ],
)


_NW = 32
_BPW = _PAD_N // _NW
_CHUNK = 80


def _gather_cols(table_t, idx):
    mesh = plsc.VectorSubcoreMesh(core_axis_name="c", subcore_axis_name="s")

    @functools.partial(
        pl.kernel, mesh=mesh,
        compiler_params=pltpu.CompilerParams(use_tc_tiling_on_sc=False),
        out_type=jax.ShapeDtypeStruct((4, _PAD_N), jnp.float32),
        scratch_types=[
            pltpu.VMEM((2, _CHUNK), jnp.int32),
            pltpu.VMEM((_CHUNK,), jnp.float32),
            pltpu.SemaphoreType.DMA,
        ],
    )
    def k(tab_hbm, idx_hbm, out_hbm, idx_v, vals_v, sem):
        wid = lax.axis_index("s") * 2 + lax.axis_index("c")
        base = wid * _BPW
        for j in range(2):
            pltpu.sync_copy(idx_hbm.at[pl.ds(base + j * _CHUNK, _CHUNK)],
                            idx_v.at[j])
        for c in range(4):
            for j in range(2):
                pltpu.async_copy(tab_hbm.at[c].at[idx_v.at[j]], vals_v, sem).wait()
                pltpu.sync_copy(vals_v,
                                out_hbm.at[c].at[pl.ds(base + j * _CHUNK, _CHUNK)])

    return k(table_t, idx)


def kernel(boxes, scores):
    keys = jnp.pad(scores.view(jnp.int32), (0, _SORT_N - scores.shape[0]),
                   constant_values=-1).reshape(_SORT_R, 128)
    ko, io = _sort_call(keys)
    idx = io.reshape(_PAD_N)
    sct = ko.reshape(1, _PAD_N).view(jnp.float32)
    rawt = _gather_cols(boxes.T, idx)
    out8 = _nms_call(rawt, sct)
    return out8[:_TOP_N, :5]


# speedup vs baseline: 1.1233x; 1.1233x over previous
import functools

import jax
import jax.numpy as jnp
from jax import lax
from jax.experimental import pallas as pl
from jax.experimental.pallas import tpu as pltpu
from jax.experimental.pallas import tpu_sc as plsc

_TOP_N = 5000
_BLK = 512
_NBLK = 10
_PAD_N = _BLK * _NBLK
_THR = 0.7

_SORT_N = 32768
_SORT_R = _SORT_N // 128
_TOPR = _PAD_N // 128


def _sort_body(key_ref, ko_ref, io_ref):
    row = lax.broadcasted_iota(jnp.int32, (_SORT_R, 128), 0)
    lane = lax.broadcasted_iota(jnp.int32, (_SORT_R, 128), 1)
    k = key_ref[...]
    v = row * 128 + lane

    def xor_rows(a, dr):
        g = _SORT_R // (2 * dr)
        a4 = a.reshape(g, 2, dr, 128)
        return jnp.concatenate([a4[:, 1:2], a4[:, 0:1]], axis=1).reshape(_SORT_R, 128)

    def xor_lanes(a, d, sel):
        return jnp.where(sel, pltpu.roll(a, 128 - d, 1), pltpu.roll(a, d, 1))

    for s in range(1, 16):
        m = 1 << s
        dirm = ((lane & m) == 0) if m <= 64 else ((row & (m >> 7)) == 0)
        d = m >> 1
        while d >= 1:
            if d >= 128:
                dr = d >> 7
                pk = xor_rows(k, dr)
                pv = xor_rows(v, dr)
                low = (row & dr) == 0
            else:
                sel = (lane & d) == 0
                pk = xor_lanes(k, d, sel)
                pv = xor_lanes(v, d, sel)
                low = sel
            self_first = (k > pk) | ((k == pk) & (v < pv))
            take_self = self_first == (low == dirm)
            k = jnp.where(take_self, k, pk)
            v = jnp.where(take_self, v, pv)
            d >>= 1
    ko_ref[...] = k[:_TOPR, :]
    io_ref[...] = v[:_TOPR, :]


_sort_call = pl.pallas_call(
    _sort_body,
    out_shape=(
        jax.ShapeDtypeStruct((_TOPR, 128), jnp.int32),
        jax.ShapeDtypeStruct((_TOPR, 128), jnp.int32),
    ),
)


def _col(v, eye):
    return lax.dot_general(eye, v, (((1,), (1,)), ((), ())),
                           preferred_element_type=jnp.float32,
                           precision=lax.Precision.HIGHEST)


def _iou(x1a, y1a, x2a, y2a, aa, x1b, y1b, x2b, y2b, ab):
    ix1 = jnp.maximum(x1a, x1b)
    iy1 = jnp.maximum(y1a, y1b)
    ix2 = jnp.minimum(x2a, x2b)
    iy2 = jnp.minimum(y2a, y2b)
    iw = jnp.maximum(ix2 - ix1, 0.0)
    ih = jnp.maximum(iy2 - iy1, 0.0)
    inter = iw * ih
    union = aa + ab - inter
    return inter / (union + 1e-8)


def _nms_body(rawt_ref, sc_ref, out_ref, c_ref, keep_ref):
    r128 = lax.broadcasted_iota(jnp.int32, (_BLK, _BLK), 0)
    c128 = lax.broadcasted_iota(jnp.int32, (_BLK, _BLK), 1)
    eye = jnp.where(r128 == c128, 1.0, 0.0).astype(jnp.float32)
    upper = r128 < c128

    rx = rawt_ref[0:1, :]
    ry = rawt_ref[1:2, :]
    rw = rawt_ref[2:3, :]
    rh = rawt_ref[3:4, :]
    cx = rx * 1000.0
    cy = ry * 1000.0
    w = rw * 200.0 + 1.0
    h = rh * 200.0 + 1.0
    c_ref[0:1, :] = cx - 0.5 * w
    c_ref[1:2, :] = cy - 0.5 * h
    c_ref[2:3, :] = cx + 0.5 * w
    c_ref[3:4, :] = cy + 0.5 * h
    c_ref[4:5, :] = (c_ref[2:3, :] - c_ref[0:1, :]) * (c_ref[3:4, :] - c_ref[1:2, :])
    keep_ref[...] = jnp.ones((1, _PAD_N), jnp.float32)

    for i in range(_NBLK):
        base = i * _BLK
        x1r = c_ref[0:1, base:base + _BLK]
        y1r = c_ref[1:2, base:base + _BLK]
        x2r = c_ref[2:3, base:base + _BLK]
        y2r = c_ref[3:4, base:base + _BLK]
        arr = c_ref[4:5, base:base + _BLK]
        x1c = _col(x1r, eye)
        y1c = _col(y1r, eye)
        x2c = _col(x2r, eye)
        y2c = _col(y2r, eye)
        arc = _col(arr, eye)

        iou_ii = _iou(x1c, y1c, x2c, y2c, arc, x1r, y1r, x2r, y2r, arr)
        E = jnp.where((iou_ii > _THR) & upper, 1.0, 0.0).astype(jnp.float32)
        inc = keep_ref[0:1, base:base + _BLK]

        def _step(k):
            cnt = lax.dot_general(k, E, (((1,), (0,)), ((), ())),
                                  preferred_element_type=jnp.float32)
            return jnp.where(cnt > 0.5, 0.0, inc)

        def _cond(carry):
            return carry[1]

        def _body(carry):
            kmid = _step(carry[0])
            knew = _step(kmid)
            changed = jnp.sum(jnp.abs(knew - kmid)) > 0.0
            return (knew, changed)

        ki, _ = lax.while_loop(_cond, _body, (inc, jnp.asarray(True)))
        keep_ref[0:1, base:base + _BLK] = ki

        if i + 1 < _NBLK:
            tail = base + _BLK
            iou_cross = _iou(x1c, y1c, x2c, y2c, arc,
                             c_ref[0:1, tail:], c_ref[1:2, tail:],
                             c_ref[2:3, tail:], c_ref[3:4, tail:],
                             c_ref[4:5, tail:])
            M = jnp.where(iou_cross > _THR, 1.0, 0.0).astype(jnp.float32)
            cnt = lax.dot_general(ki, M, (((1,), (0,)), ((), ())),
                                  preferred_element_type=jnp.float32)
            keep_ref[0:1, tail:] = jnp.where(cnt > 0.5, 0.0,
                                             keep_ref[0:1, tail:])

    k = keep_ref[...]
    o8 = jnp.concatenate([
        c_ref[0:4, :] * k,
        sc_ref[...] * k,
        jnp.zeros((3, _PAD_N), jnp.float32),
    ], axis=0)
    for i in range(_NBLK):
        base = i * _BLK
        out_ref[base:base + _BLK, :] = lax.dot_general(
            eye, o8[:, base:base + _BLK], (((1,), (1,)), ((), ())),
            preferred_element_type=jnp.float32,
            precision=lax.Precision.HIGHEST)


_nms_call = pl.pallas_call(
    _nms_body,
    out_shape=jax.ShapeDtypeStruct((_PAD_N, 8), jnp.float32),
    scratch_shapes=[
        pltpu.VMEM((8, _PAD_N), jnp.float32),
        pltpu.VMEM((1, _PAD_N), jnp.float32),
    ],
)


_NW = 32
_BPW = _PAD_N // _NW
_CHUNK = 80


def _gather_cols(table_t, idx):
    mesh = plsc.VectorSubcoreMesh(core_axis_name="c", subcore_axis_name="s")

    @functools.partial(
        pl.kernel, mesh=mesh,
        compiler_params=pltpu.CompilerParams(use_tc_tiling_on_sc=False),
        out_type=jax.ShapeDtypeStruct((4, _PAD_N), jnp.float32),
        scratch_types=[
            pltpu.VMEM((2, _CHUNK), jnp.int32),
            pltpu.VMEM((_CHUNK,), jnp.float32),
            pltpu.SemaphoreType.DMA,
        ],
    )
    def k(tab_hbm, idx_hbm, out_hbm, idx_v, vals_v, sem):
        wid = lax.axis_index("s") * 2 + lax.axis_index("c")
        base = wid * _BPW
        for j in range(2):
            pltpu.sync_copy(idx_hbm.at[pl.ds(base + j * _CHUNK, _CHUNK)],
                            idx_v.at[j])
        for c in range(4):
            for j in range(2):
                pltpu.async_copy(tab_hbm.at[c].at[idx_v.at[j]], vals_v, sem).wait()
                pltpu.sync_copy(vals_v,
                                out_hbm.at[c].at[pl.ds(base + j * _CHUNK, _CHUNK)])

    return k(table_t, idx)


def kernel(boxes, scores):
    keys = jnp.pad(scores.view(jnp.int32), (0, _SORT_N - scores.shape[0]),
                   constant_values=-1).reshape(_SORT_R, 128)
    ko, io = _sort_call(keys)
    idx = io.reshape(_PAD_N)
    sct = ko.reshape(1, _PAD_N).view(jnp.float32)
    rawt = _gather_cols(boxes.T, idx)
    out8 = _nms_call(rawt, sct)
    return out8[:_TOP_N, :5]
